# Initial kernel scaffold; baseline (speedup 1.0000x reference)
import functools

import jax
import jax.numpy as jnp
from jax import lax
from jax.experimental import pallas as pl
from jax.experimental.pallas import tpu as pltpu
from jax.experimental.pallas import tpu_sc as plsc

N = 100000
F = 128
S = 64
NC = 2
NS = 16
CHUNK = 128
NFULL = N // CHUNK
REM = N - NFULL * CHUNK
FH = F // NC
ITERS = -(-NFULL // NS)

_mesh = plsc.VectorSubcoreMesh(core_axis_name="c", subcore_axis_name="s")


@functools.partial(
    pl.kernel,
    out_type=jax.ShapeDtypeStruct((S, F), jnp.float32),
    mesh=_mesh,
    scratch_types=[
        pltpu.VMEM((CHUNK,), jnp.int32),
        pltpu.VMEM((CHUNK, FH), jnp.float32),
        pltpu.VMEM((REM,), jnp.int32),
        pltpu.VMEM((REM, FH), jnp.float32),
        pltpu.VMEM((S, FH), jnp.float32),
        pltpu.VMEM_SHARED((S, FH), jnp.float32),
    ],
)
def _seg_sum(x_hbm, b_hbm, out_hbm, idx_v, rows_v, idx_r, rows_r, zbuf,
             acc_sh):
    core = lax.axis_index("c")
    sub = lax.axis_index("s")
    col0 = core * FH

    @pl.when(sub == 0)
    def _():
        zeros16 = jnp.zeros((16,), jnp.float32)

        def zrow(r, carry):
            for k in range(FH // 16):
                zbuf[r, pl.ds(k * 16, 16)] = zeros16
            return carry

        lax.fori_loop(0, S, zrow, 0)
        pltpu.sync_copy(zbuf, acc_sh)

    plsc.subcore_barrier()

    def body(i, carry):
        c = sub + i * NS

        @pl.when(c < NFULL)
        def _():
            r0 = c * CHUNK
            pltpu.sync_copy(b_hbm.at[pl.ds(r0, CHUNK)], idx_v)
            pltpu.sync_copy(
                x_hbm.at[pl.ds(r0, CHUNK), pl.ds(col0, FH)], rows_v)
            pltpu.sync_copy(rows_v, acc_sh.at[idx_v], add=True)

        return carry

    lax.fori_loop(0, ITERS, body, 0)

    @pl.when(sub == NS - 1)
    def _():
        r0 = NFULL * CHUNK
        pltpu.sync_copy(b_hbm.at[pl.ds(r0, REM)], idx_r)
        pltpu.sync_copy(x_hbm.at[pl.ds(r0, REM), pl.ds(col0, FH)], rows_r)
        pltpu.sync_copy(rows_r, acc_sh.at[idx_r], add=True)

    plsc.subcore_barrier()

    @pl.when(sub == 0)
    def _():
        pltpu.sync_copy(acc_sh, out_hbm.at[pl.ds(0, S), pl.ds(col0, FH)])


def kernel(x, batch, batch_size):
    del batch_size
    return _seg_sum(x, batch.astype(jnp.int32))


# speedup vs baseline: 3.2858x; 3.2858x over previous
import functools

import jax
import jax.numpy as jnp
from jax import lax
from jax.experimental import pallas as pl
from jax.experimental.pallas import tpu as pltpu
from jax.experimental.pallas import tpu_sc as plsc

N = 100000
F = 128
S = 64
NC = 2
NS = 16
CHUNK = 128
NFULL = N // CHUNK
REM = N - NFULL * CHUNK
FH = F // NC
ITERS = -(-NFULL // NS)

_mesh = plsc.VectorSubcoreMesh(core_axis_name="c", subcore_axis_name="s")


@functools.partial(
    pl.kernel,
    out_type=jax.ShapeDtypeStruct((S, F), jnp.float32),
    mesh=_mesh,
    scratch_types=[
        pltpu.VMEM((CHUNK,), jnp.int32),
        pltpu.VMEM((CHUNK, FH), jnp.float32),
        pltpu.VMEM((REM,), jnp.int32),
        pltpu.VMEM((REM, FH), jnp.float32),
        pltpu.VMEM((S, FH), jnp.float32),
        pltpu.VMEM_SHARED((S, FH), jnp.float32),
    ],
    compiler_params=pltpu.CompilerParams(use_tc_tiling_on_sc=False),
)
def _seg_sum(x_hbm, b_hbm, out_hbm, idx_v, rows_v, idx_r, rows_r, zbuf,
             acc_sh):
    core = lax.axis_index("c")
    sub = lax.axis_index("s")
    col0 = core * FH

    @pl.when(sub == 0)
    def _():
        zeros16 = jnp.zeros((16,), jnp.float32)

        def zrow(r, carry):
            for k in range(FH // 16):
                zbuf[r, pl.ds(k * 16, 16)] = zeros16
            return carry

        lax.fori_loop(0, S, zrow, 0)
        pltpu.sync_copy(zbuf, acc_sh)

    plsc.subcore_barrier()

    def body(i, carry):
        c = sub + i * NS

        @pl.when(c < NFULL)
        def _():
            r0 = c * CHUNK
            pltpu.sync_copy(b_hbm.at[pl.ds(r0, CHUNK)], idx_v)
            pltpu.sync_copy(
                x_hbm.at[pl.ds(r0, CHUNK), pl.ds(col0, FH)], rows_v)
            pltpu.sync_copy(rows_v, acc_sh.at[idx_v], add=True)

        return carry

    lax.fori_loop(0, ITERS, body, 0)

    @pl.when(sub == NS - 1)
    def _():
        r0 = NFULL * CHUNK
        pltpu.sync_copy(b_hbm.at[pl.ds(r0, REM)], idx_r)
        pltpu.sync_copy(x_hbm.at[pl.ds(r0, REM), pl.ds(col0, FH)], rows_r)
        pltpu.sync_copy(rows_r, acc_sh.at[idx_r], add=True)

    plsc.subcore_barrier()

    @pl.when(sub == 0)
    def _():
        pltpu.sync_copy(acc_sh, out_hbm.at[pl.ds(0, S), pl.ds(col0, FH)])


def kernel(x, batch, batch_size):
    del batch_size
    return _seg_sum(x, batch.astype(jnp.int32))


# speedup vs baseline: 4.7722x; 1.4524x over previous
import functools

import jax
import jax.numpy as jnp
from jax import lax
from jax.experimental import pallas as pl
from jax.experimental.pallas import tpu as pltpu
from jax.experimental.pallas import tpu_sc as plsc

N = 100000
F = 128
S = 64
NC = 2
NS = 16
CHUNK = 128
NFULL = N // CHUNK
REM = N - NFULL * CHUNK
FH = F // NC
GROUP = 4
GROWS = GROUP * CHUNK
NGROUPS = 12
NEXTRA = NFULL - NS * NGROUPS * GROUP

_mesh = plsc.VectorSubcoreMesh(core_axis_name="c", subcore_axis_name="s")


@functools.partial(
    pl.kernel,
    out_type=jax.ShapeDtypeStruct((S, F), jnp.float32),
    mesh=_mesh,
    scratch_types=[
        pltpu.VMEM((GROWS, FH), jnp.float32),
        pltpu.VMEM((GROWS, FH), jnp.float32),
        pltpu.VMEM((GROUP, CHUNK), jnp.int32),
        pltpu.VMEM((GROUP, CHUNK), jnp.int32),
        pltpu.VMEM((CHUNK, FH), jnp.float32),
        pltpu.VMEM((1, CHUNK), jnp.int32),
        pltpu.VMEM((REM, FH), jnp.float32),
        pltpu.VMEM((REM,), jnp.int32),
        pltpu.VMEM((S, FH), jnp.float32),
        pltpu.VMEM_SHARED((S, FH), jnp.float32),
        pltpu.SemaphoreType.DMA,
        pltpu.SemaphoreType.DMA,
        pltpu.SemaphoreType.DMA,
    ],
    compiler_params=pltpu.CompilerParams(use_tc_tiling_on_sc=False),
)
def _seg_sum(x_hbm, b2_hbm, brem_hbm, out_hbm, xb0, xb1, ib0, ib1, xbe, ibe,
             xbr, ibr, zbuf, acc_sh, sem0, sem1, sem_s):
    core = lax.axis_index("c")
    sub = lax.axis_index("s")
    col0 = core * FH
    c0 = sub * (NGROUPS * GROUP) + jnp.minimum(sub, NEXTRA)
    r0 = c0 * CHUNK

    def start_group(g, xb, ib, sem):
        pltpu.async_copy(
            x_hbm.at[pl.ds(r0 + g * GROWS, GROWS), pl.ds(col0, FH)], xb, sem)
        pltpu.async_copy(b2_hbm.at[pl.ds(c0 + g * GROUP, GROUP)], ib, sem)

    def wait_group(xb, ib, sem):
        pltpu.make_async_copy(
            x_hbm.at[pl.ds(0, GROWS), pl.ds(col0, FH)], xb, sem).wait()
        pltpu.make_async_copy(b2_hbm.at[pl.ds(0, GROUP)], ib, sem).wait()

    def scatter_group(xb, ib):
        descs = [
            pltpu.async_copy(
                xb.at[pl.ds(j * CHUNK, CHUNK)], acc_sh.at[ib.at[j]], sem_s,
                add=True)
            for j in range(GROUP)
        ]
        for d in descs:
            d.wait()

    start_group(0, xb0, ib0, sem0)
    start_group(1, xb1, ib1, sem1)

    @pl.when(sub == 0)
    def _():
        zeros16 = jnp.zeros((16,), jnp.float32)

        def zrow(r, carry):
            for k in range(FH // 16):
                zbuf[r, pl.ds(k * 16, 16)] = zeros16
            return carry

        lax.fori_loop(0, S, zrow, 0)
        pltpu.sync_copy(zbuf, acc_sh)

    plsc.subcore_barrier()

    def body(i, carry):
        for b, (xb, ib, sem) in enumerate(
                ((xb0, ib0, sem0), (xb1, ib1, sem1))):
            g = 2 * i + b
            wait_group(xb, ib, sem)
            scatter_group(xb, ib)

            @pl.when(g + 2 < NGROUPS)
            def _():
                start_group(g + 2, xb, ib, sem)

        return carry

    lax.fori_loop(0, NGROUPS // 2, body, 0)

    @pl.when(sub < NEXTRA)
    def _():
        pltpu.sync_copy(
            x_hbm.at[pl.ds(r0 + NGROUPS * GROWS, CHUNK), pl.ds(col0, FH)],
            xbe)
        pltpu.sync_copy(b2_hbm.at[pl.ds(c0 + NGROUPS * GROUP, 1)], ibe)
        pltpu.sync_copy(xbe, acc_sh.at[ibe.at[0]], add=True)

    @pl.when(sub == NS - 1)
    def _():
        pltpu.sync_copy(
            x_hbm.at[pl.ds(NFULL * CHUNK, REM), pl.ds(col0, FH)], xbr)
        pltpu.sync_copy(brem_hbm, ibr)
        pltpu.sync_copy(xbr, acc_sh.at[ibr], add=True)

    plsc.subcore_barrier()

    @pl.when(sub == 0)
    def _():
        pltpu.sync_copy(acc_sh, out_hbm.at[pl.ds(0, S), pl.ds(col0, FH)])


def kernel(x, batch, batch_size):
    del batch_size
    b = batch.astype(jnp.int32)
    b2 = b[:NFULL * CHUNK].reshape(NFULL, CHUNK)
    brem = b[NFULL * CHUNK:]
    return _seg_sum(x, b2, brem)
